# baseline (speedup 1.0000x reference)
import functools

import jax
import jax.numpy as jnp
from jax import lax
from jax.experimental import pallas as pl
from jax.experimental.pallas import tpu as pltpu
from jax.experimental.pallas import tpu_sc as plsc

N = 10000
E = 320000
D_IN = 128
HIDDEN = 256
NUM_CLASSES = 64

NC = 2
NS = 16
CHUNK = 128

K_GRP = 32
E_PAD = 327680
K_ALL = E_PAD // (NS * CHUNK)
K_HALF = E_PAD // (NC * NS * CHUNK)
NGRP = K_ALL // K_GRP
K_GRP2 = 40

NR = 10112
RPT = NR // NS
BN = 400
GRID = N // BN

_mesh = lambda: plsc.VectorSubcoreMesh(
    core_axis_name="c", subcore_axis_name="s", num_cores=NC, num_subcores=NS
)


def _zero_rows(buf, nrows, ncols):
    @pl.loop(0, nrows)
    def _(i):
        @pl.loop(0, ncols // 16)
        def _(k):
            buf[i, pl.ds(k * 16, 16)] = jnp.zeros((16,), jnp.float32)


def _zero_acc_slice(zbuf, acc, base):
    full = RPT // CHUNK
    rem = RPT - full * CHUNK
    for k in range(full):
        pltpu.sync_copy(zbuf, acc.at[pl.ds(base + k * CHUNK, CHUNK)])
    if rem:
        pltpu.sync_copy(zbuf.at[pl.ds(0, rem)], acc.at[pl.ds(base + full * CHUNK, rem)])


def _deg_body(dst_hbm, out_hbm, buf, dstbuf, acc):
    c = lax.axis_index("c")
    sid = lax.axis_index("s")
    base = sid * RPT
    _zero_rows(buf, CHUNK, 128)
    _zero_acc_slice(buf, acc, base)
    @pl.loop(0, CHUNK)
    def _(i):
        @pl.loop(0, 8)
        def _(k):
            buf[i, pl.ds(k * 16, 16)] = jnp.full((16,), 1.0, jnp.float32)
    pltpu.sync_copy(dst_hbm.at[c, sid], dstbuf)
    plsc.subcore_barrier()
    @pl.loop(0, K_HALF)
    def _(j):
        pltpu.sync_copy(buf, acc.at[dstbuf.at[j]], add=True)
    plsc.subcore_barrier()
    pltpu.sync_copy(acc.at[pl.ds(base, RPT)], out_hbm.at[c, pl.ds(base, RPT)])


@functools.lru_cache(maxsize=None)
def _get_deg_kernel():
    return pl.kernel(
        _deg_body,
        out_type=jax.ShapeDtypeStruct((NC, NR, 128), jnp.float32),
        mesh=_mesh(),
        scratch_types=[
            pltpu.VMEM((CHUNK, 128), jnp.float32),
            pltpu.VMEM((K_HALF, CHUNK), jnp.int32),
            pltpu.VMEM_SHARED((NR, 128), jnp.float32),
        ],
    )


def _segsum_body(dh, s_hbm, src_hbm, dst_hbm, out_hbm, gbuf, dstbuf, rows0, rows1,
                 acc, sem0, sem1, ssem0, ssem1):
    c = lax.axis_index("c")
    sid = lax.axis_index("s")
    base = sid * RPT
    _zero_rows(rows0, CHUNK, dh)
    _zero_acc_slice(rows0, acc, base)
    plsc.subcore_barrier()
    @pl.loop(0, NGRP)
    def _(g):
        pltpu.sync_copy(src_hbm.at[sid, pl.ds(g * K_GRP, K_GRP)], gbuf)
        pltpu.sync_copy(dst_hbm.at[sid, pl.ds(g * K_GRP, K_GRP)], dstbuf)
        @pl.loop(0, K_GRP)
        def _(j):
            @pl.loop(0, CHUNK // 16)
            def _(i):
                v = gbuf[j, pl.ds(i * 16, 16)]
                gbuf[j, pl.ds(i * 16, 16)] = v + v + c
        pltpu.async_copy(s_hbm.at[gbuf.at[0]], rows0, sem0)
        @pl.loop(0, K_GRP, step=2)
        def _(j):
            j1 = j + 1
            j2 = jnp.minimum(j + 2, K_GRP - 1)
            pltpu.async_copy(s_hbm.at[gbuf.at[j1]], rows1, sem1)
            pltpu.make_async_copy(s_hbm.at[gbuf.at[j]], rows0, sem0).wait()
            pltpu.sync_copy(rows0, acc.at[dstbuf.at[j]], add=True)
            pltpu.async_copy(s_hbm.at[gbuf.at[j2]], rows0, sem0)
            pltpu.make_async_copy(s_hbm.at[gbuf.at[j1]], rows1, sem1).wait()
            pltpu.sync_copy(rows1, acc.at[dstbuf.at[j1]], add=True)
        pltpu.make_async_copy(s_hbm.at[gbuf.at[K_GRP - 1]], rows0, sem0).wait()
    plsc.subcore_barrier()
    pltpu.sync_copy(acc.at[pl.ds(base, RPT)], out_hbm.at[c, pl.ds(base, RPT)])


def _segsum2_body(s_hbm, src_hbm, dst_hbm, out_hbm, srcbuf, dstbuf, rows0, rows1,
                  acc, sem0, sem1, ssem0, ssem1):
    c = lax.axis_index("c")
    sid = lax.axis_index("s")
    base = sid * RPT
    _zero_rows(rows0, CHUNK, 128)
    _zero_acc_slice(rows0, acc, base)
    plsc.subcore_barrier()
    @pl.loop(0, K_HALF // K_GRP2)
    def _(g):
        pltpu.sync_copy(src_hbm.at[c, sid, pl.ds(g * K_GRP2, K_GRP2)], srcbuf)
        pltpu.sync_copy(dst_hbm.at[c, sid, pl.ds(g * K_GRP2, K_GRP2)], dstbuf)
        pltpu.async_copy(s_hbm.at[srcbuf.at[0]], rows0, sem0)
        @pl.loop(0, K_GRP2, step=2)
        def _(j):
            j1 = j + 1
            j2 = jnp.minimum(j + 2, K_GRP2 - 1)
            pltpu.async_copy(s_hbm.at[srcbuf.at[j1]], rows1, sem1)
            pltpu.make_async_copy(s_hbm.at[srcbuf.at[j]], rows0, sem0).wait()
            pltpu.sync_copy(rows0, acc.at[dstbuf.at[j]], add=True)
            pltpu.async_copy(s_hbm.at[srcbuf.at[j2]], rows0, sem0)
            pltpu.make_async_copy(s_hbm.at[srcbuf.at[j1]], rows1, sem1).wait()
            pltpu.sync_copy(rows1, acc.at[dstbuf.at[j1]], add=True)
        pltpu.make_async_copy(s_hbm.at[srcbuf.at[K_GRP2 - 1]], rows0, sem0).wait()
    plsc.subcore_barrier()
    pltpu.sync_copy(acc.at[pl.ds(base, RPT)], out_hbm.at[c, pl.ds(base, RPT)])


@functools.lru_cache(maxsize=None)
def _get_segsum2():
    return pl.kernel(
        _segsum2_body,
        out_type=jax.ShapeDtypeStruct((NC, NR, 128), jnp.float32),
        mesh=_mesh(),
        scratch_types=[
            pltpu.VMEM((K_GRP2, CHUNK), jnp.int32),
            pltpu.VMEM((K_GRP2, CHUNK), jnp.int32),
            pltpu.VMEM((CHUNK, 128), jnp.float32),
            pltpu.VMEM((CHUNK, 128), jnp.float32),
            pltpu.VMEM_SHARED((NR, 128), jnp.float32),
            pltpu.SemaphoreType.DMA,
            pltpu.SemaphoreType.DMA,
            pltpu.SemaphoreType.DMA,
            pltpu.SemaphoreType.DMA,
        ],
    )


@functools.lru_cache(maxsize=None)
def _make_segsum(dh):
    return pl.kernel(
        functools.partial(_segsum_body, dh),
        out_type=jax.ShapeDtypeStruct((NC, NR, dh), jnp.float32),
        mesh=_mesh(),
        scratch_types=[
            pltpu.VMEM((K_GRP, CHUNK), jnp.int32),
            pltpu.VMEM((K_GRP, CHUNK), jnp.int32),
            pltpu.VMEM((CHUNK, dh), jnp.float32),
            pltpu.VMEM((CHUNK, dh), jnp.float32),
            pltpu.VMEM_SHARED((NR, dh), jnp.float32),
            pltpu.SemaphoreType.DMA,
            pltpu.SemaphoreType.DMA,
            pltpu.SemaphoreType.DMA,
            pltpu.SemaphoreType.DMA,
        ],
    )


def _dense1a_body(x_ref, w_ref, xw_ref):
    xw_ref[...] = jnp.dot(x_ref[...], w_ref[...], preferred_element_type=jnp.float32)


_dense1a = pl.pallas_call(
    _dense1a_body,
    grid=(GRID,),
    in_specs=[
        pl.BlockSpec((BN, D_IN), lambda j: (j, 0)),
        pl.BlockSpec((D_IN, HIDDEN), lambda j: (0, 0)),
    ],
    out_specs=pl.BlockSpec((BN, HIDDEN), lambda j: (j, 0)),
    out_shape=jax.ShapeDtypeStruct((N, HIDDEN), jnp.float32),
)


def _dense1b_body(degp_ref, xw_ref, s_ref, dinv_ref):
    deg = degp_ref[0][:, :16] + degp_ref[1][:, :16] + 1.0
    dinv = lax.rsqrt(deg)
    dinv_ref[...] = dinv
    s_ref[...] = xw_ref[...] * dinv[:, 0:1]


_dense1b = pl.pallas_call(
    _dense1b_body,
    grid=(GRID,),
    in_specs=[
        pl.BlockSpec((NC, BN, 128), lambda j: (0, j, 0)),
        pl.BlockSpec((BN, HIDDEN), lambda j: (j, 0)),
    ],
    out_specs=[
        pl.BlockSpec((BN, HIDDEN), lambda j: (j, 0)),
        pl.BlockSpec((BN, 16), lambda j: (j, 0)),
    ],
    out_shape=[
        jax.ShapeDtypeStruct((N, HIDDEN), jnp.float32),
        jax.ShapeDtypeStruct((N, 16), jnp.float32),
    ],
)


def _dense2_body(t1_ref, s1_ref, dinv_ref, b1_ref, w2_ref, s2_ref):
    di = dinv_ref[...][:, 0:1]
    t = jnp.concatenate([t1_ref[0], t1_ref[1]], axis=1)
    h = jnp.maximum(di * (t + s1_ref[...]) + b1_ref[...], 0.0)
    s2_ref[...] = jnp.dot(h, w2_ref[...], preferred_element_type=jnp.float32) * di


_dense2 = pl.pallas_call(
    _dense2_body,
    grid=(GRID,),
    in_specs=[
        pl.BlockSpec((NC, BN, HIDDEN // 2), lambda j: (0, j, 0)),
        pl.BlockSpec((BN, HIDDEN), lambda j: (j, 0)),
        pl.BlockSpec((BN, 16), lambda j: (j, 0)),
        pl.BlockSpec((1, HIDDEN), lambda j: (0, 0)),
        pl.BlockSpec((HIDDEN, NUM_CLASSES), lambda j: (0, 0)),
    ],
    out_specs=pl.BlockSpec((BN, NUM_CLASSES), lambda j: (j, 0)),
    out_shape=jax.ShapeDtypeStruct((N, NUM_CLASSES), jnp.float32),
)


def _dense3_body(t2_ref, s2_ref, dinv_ref, b2_ref, o_ref):
    di = dinv_ref[...][:, 0:1]
    t = t2_ref[0][:, :NUM_CLASSES] + t2_ref[1][:, :NUM_CLASSES]
    o_ref[...] = jnp.maximum(di * (t + s2_ref[...]) + b2_ref[...], 0.0)


_dense3 = pl.pallas_call(
    _dense3_body,
    grid=(GRID,),
    in_specs=[
        pl.BlockSpec((NC, BN, 128), lambda j: (0, j, 0)),
        pl.BlockSpec((BN, NUM_CLASSES), lambda j: (j, 0)),
        pl.BlockSpec((BN, 16), lambda j: (j, 0)),
        pl.BlockSpec((1, NUM_CLASSES), lambda j: (0, 0)),
    ],
    out_specs=pl.BlockSpec((BN, NUM_CLASSES), lambda j: (j, 0)),
    out_shape=jax.ShapeDtypeStruct((N, NUM_CLASSES), jnp.float32),
)


@jax.jit
def kernel(x, edge_index, W1, b1, W2, b2):
    src = edge_index[0]
    dst = edge_index[1]
    pad = E_PAD - E
    pad_ar = jnp.arange(pad, dtype=jnp.int32)
    srcp = jnp.concatenate([src, pad_ar % N])
    dstp = jnp.concatenate([dst, N + pad_ar % (NR - N)])
    src_all = srcp.reshape(NS, K_ALL, CHUNK)
    dst_all = dstp.reshape(NS, K_ALL, CHUNK)
    src_half = srcp.reshape(NC, NS, K_HALF, CHUNK)
    dst_half = dstp.reshape(NC, NS, K_HALF, CHUNK)

    xw1 = _dense1a(x, W1)
    degp = _get_deg_kernel()(dst_half)
    s1, dinv16 = _dense1b(degp, xw1)
    t1 = _make_segsum(HIDDEN // 2)(s1.reshape(2 * N, HIDDEN // 2), src_all, dst_all)
    s2 = _dense2(t1, s1, dinv16, b1.reshape(1, HIDDEN), W2)
    s2p = jnp.pad(s2, ((0, 0), (0, 128 - NUM_CLASSES)))
    t2 = _get_segsum2()(s2p, src_half, dst_half)
    out = _dense3(t2, s2, dinv16, b2.reshape(1, NUM_CLASSES))
    return out


# speedup vs baseline: 1.2945x; 1.2945x over previous
import functools

import jax
import jax.numpy as jnp
from jax import lax
from jax.experimental import pallas as pl
from jax.experimental.pallas import tpu as pltpu
from jax.experimental.pallas import tpu_sc as plsc

N = 10000
E = 320000
D_IN = 128
HIDDEN = 256
NUM_CLASSES = 64

NC = 2
NS = 16
CHUNK = 128

E_PAD = 327680
K_HALF = E_PAD // (NC * NS * CHUNK)
K_GRP2 = 40

NR = 10112
RPT = NR // NS
BN = 400
GRID = N // BN

_mesh = lambda: plsc.VectorSubcoreMesh(
    core_axis_name="c", subcore_axis_name="s", num_cores=NC, num_subcores=NS
)


def _zero_rows(buf, nrows, ncols):
    @pl.loop(0, nrows)
    def _(i):
        @pl.loop(0, ncols // 16)
        def _(k):
            buf[i, pl.ds(k * 16, 16)] = jnp.zeros((16,), jnp.float32)


def _zero_acc_slice(zbuf, acc, base):
    full = RPT // CHUNK
    rem = RPT - full * CHUNK
    for k in range(full):
        pltpu.sync_copy(zbuf, acc.at[pl.ds(base + k * CHUNK, CHUNK)])
    if rem:
        pltpu.sync_copy(zbuf.at[pl.ds(0, rem)], acc.at[pl.ds(base + full * CHUNK, rem)])


def _deg_body(dst_hbm, out_hbm, buf, dstbuf, acc):
    c = lax.axis_index("c")
    sid = lax.axis_index("s")
    base = sid * RPT
    _zero_rows(buf, CHUNK, 128)
    _zero_acc_slice(buf, acc, base)
    @pl.loop(0, CHUNK)
    def _(i):
        @pl.loop(0, 8)
        def _(k):
            buf[i, pl.ds(k * 16, 16)] = jnp.full((16,), 1.0, jnp.float32)
    pltpu.sync_copy(dst_hbm.at[c, sid], dstbuf)
    plsc.subcore_barrier()
    @pl.loop(0, K_HALF)
    def _(j):
        pltpu.sync_copy(buf, acc.at[dstbuf.at[j]], add=True)
    plsc.subcore_barrier()
    pltpu.sync_copy(acc.at[pl.ds(base, RPT)], out_hbm.at[c, pl.ds(base, RPT)])


@functools.lru_cache(maxsize=None)
def _get_deg_kernel():
    return pl.kernel(
        _deg_body,
        out_type=jax.ShapeDtypeStruct((NC, NR, 128), jnp.float32),
        mesh=_mesh(),
        scratch_types=[
            pltpu.VMEM((CHUNK, 128), jnp.float32),
            pltpu.VMEM((K_HALF, CHUNK), jnp.int32),
            pltpu.VMEM_SHARED((NR, 128), jnp.float32),
        ],
    )


def _segsum2_body(s_hbm, src_hbm, dst_hbm, out_hbm, srcbuf, dstbuf, rows0, rows1,
                  acc, sem0, sem1, ssem0, ssem1):
    c = lax.axis_index("c")
    sid = lax.axis_index("s")
    base = sid * RPT
    _zero_rows(rows0, CHUNK, 128)
    _zero_acc_slice(rows0, acc, base)
    plsc.subcore_barrier()
    @pl.loop(0, K_HALF // K_GRP2)
    def _(g):
        pltpu.sync_copy(src_hbm.at[c, sid, pl.ds(g * K_GRP2, K_GRP2)], srcbuf)
        pltpu.sync_copy(dst_hbm.at[c, sid, pl.ds(g * K_GRP2, K_GRP2)], dstbuf)
        pltpu.async_copy(s_hbm.at[srcbuf.at[0]], rows0, sem0)
        @pl.loop(0, K_GRP2, step=2)
        def _(j):
            j1 = j + 1
            j2 = jnp.minimum(j + 2, K_GRP2 - 1)
            pltpu.async_copy(s_hbm.at[srcbuf.at[j1]], rows1, sem1)
            pltpu.make_async_copy(s_hbm.at[srcbuf.at[j]], rows0, sem0).wait()
            pltpu.sync_copy(rows0, acc.at[dstbuf.at[j]], add=True)
            pltpu.async_copy(s_hbm.at[srcbuf.at[j2]], rows0, sem0)
            pltpu.make_async_copy(s_hbm.at[srcbuf.at[j1]], rows1, sem1).wait()
            pltpu.sync_copy(rows1, acc.at[dstbuf.at[j1]], add=True)
        pltpu.make_async_copy(s_hbm.at[srcbuf.at[K_GRP2 - 1]], rows0, sem0).wait()
    plsc.subcore_barrier()
    pltpu.sync_copy(acc.at[pl.ds(base, RPT)], out_hbm.at[c, pl.ds(base, RPT)])


@functools.lru_cache(maxsize=None)
def _get_segsum2():
    return pl.kernel(
        _segsum2_body,
        out_type=jax.ShapeDtypeStruct((NC, NR, 128), jnp.float32),
        mesh=_mesh(),
        scratch_types=[
            pltpu.VMEM((K_GRP2, CHUNK), jnp.int32),
            pltpu.VMEM((K_GRP2, CHUNK), jnp.int32),
            pltpu.VMEM((CHUNK, 128), jnp.float32),
            pltpu.VMEM((CHUNK, 128), jnp.float32),
            pltpu.VMEM_SHARED((NR, 128), jnp.float32),
            pltpu.SemaphoreType.DMA,
            pltpu.SemaphoreType.DMA,
            pltpu.SemaphoreType.DMA,
            pltpu.SemaphoreType.DMA,
        ],
    )


def _dense1_body(degp_ref, x_ref, xs_ref, dinv_ref):
    deg = degp_ref[0][:, :16] + degp_ref[1][:, :16] + 1.0
    dinv = lax.rsqrt(deg)
    dinv_ref[...] = dinv
    xs_ref[...] = x_ref[...] * dinv[:, 0:1]


_dense1 = pl.pallas_call(
    _dense1_body,
    grid=(GRID,),
    in_specs=[
        pl.BlockSpec((NC, BN, 128), lambda j: (0, j, 0)),
        pl.BlockSpec((BN, D_IN), lambda j: (j, 0)),
    ],
    out_specs=[
        pl.BlockSpec((BN, D_IN), lambda j: (j, 0)),
        pl.BlockSpec((BN, 16), lambda j: (j, 0)),
    ],
    out_shape=[
        jax.ShapeDtypeStruct((N, D_IN), jnp.float32),
        jax.ShapeDtypeStruct((N, 16), jnp.float32),
    ],
)


def _dense2_body(t1_ref, x_ref, dinv_ref, b1_ref, w1_ref, w2_ref, s2_ref):
    di = dinv_ref[...][:, 0:1]
    z = t1_ref[0] + t1_ref[1] + x_ref[...] * di
    h = jnp.maximum(
        di * jnp.dot(z, w1_ref[...], preferred_element_type=jnp.float32)
        + b1_ref[...], 0.0)
    s2_ref[...] = jnp.dot(h, w2_ref[...], preferred_element_type=jnp.float32) * di


_dense2 = pl.pallas_call(
    _dense2_body,
    grid=(GRID,),
    in_specs=[
        pl.BlockSpec((NC, BN, 128), lambda j: (0, j, 0)),
        pl.BlockSpec((BN, D_IN), lambda j: (j, 0)),
        pl.BlockSpec((BN, 16), lambda j: (j, 0)),
        pl.BlockSpec((1, HIDDEN), lambda j: (0, 0)),
        pl.BlockSpec((D_IN, HIDDEN), lambda j: (0, 0)),
        pl.BlockSpec((HIDDEN, NUM_CLASSES), lambda j: (0, 0)),
    ],
    out_specs=pl.BlockSpec((BN, NUM_CLASSES), lambda j: (j, 0)),
    out_shape=jax.ShapeDtypeStruct((N, NUM_CLASSES), jnp.float32),
)


def _dense3_body(t2_ref, s2_ref, dinv_ref, b2_ref, o_ref):
    di = dinv_ref[...][:, 0:1]
    t = t2_ref[0][:, :NUM_CLASSES] + t2_ref[1][:, :NUM_CLASSES]
    o_ref[...] = jnp.maximum(di * (t + s2_ref[...]) + b2_ref[...], 0.0)


_dense3 = pl.pallas_call(
    _dense3_body,
    grid=(GRID,),
    in_specs=[
        pl.BlockSpec((NC, BN, 128), lambda j: (0, j, 0)),
        pl.BlockSpec((BN, NUM_CLASSES), lambda j: (j, 0)),
        pl.BlockSpec((BN, 16), lambda j: (j, 0)),
        pl.BlockSpec((1, NUM_CLASSES), lambda j: (0, 0)),
    ],
    out_specs=pl.BlockSpec((BN, NUM_CLASSES), lambda j: (j, 0)),
    out_shape=jax.ShapeDtypeStruct((N, NUM_CLASSES), jnp.float32),
)


@jax.jit
def kernel(x, edge_index, W1, b1, W2, b2):
    src = edge_index[0]
    dst = edge_index[1]
    pad = E_PAD - E
    pad_ar = jnp.arange(pad, dtype=jnp.int32)
    srcp = jnp.concatenate([src, pad_ar % N])
    dstp = jnp.concatenate([dst, N + pad_ar % (NR - N)])
    src_half = srcp.reshape(NC, NS, K_HALF, CHUNK)
    dst_half = dstp.reshape(NC, NS, K_HALF, CHUNK)

    degp = _get_deg_kernel()(dst_half)
    xs, dinv16 = _dense1(degp, x)
    t1 = _get_segsum2()(xs, src_half, dst_half)
    s2 = _dense2(t1, x, dinv16, b1.reshape(1, HIDDEN), W1, W2)
    s2p = jnp.pad(s2, ((0, 0), (0, 128 - NUM_CLASSES)))
    t2 = _get_segsum2()(s2p, src_half, dst_half)
    out = _dense3(t2, s2, dinv16, b2.reshape(1, NUM_CLASSES))
    return out
